# baseline (speedup 1.0000x reference)
import functools

import jax
import jax.numpy as jnp
from jax import lax
from jax.experimental import pallas as pl
from jax.experimental.pallas import tpu as pltpu
from jax.experimental.pallas import tpu_sc as plsc

NC, NS, L = 2, 16, 16
NW = NC * NS
R = 8


def _sc_body(x_hbm, pe_hbm, o_hbm, xb, peb, ob, xin_sem, out_sem, pe_sem,
             *, rows_w, batch, nch, d):
    wid = lax.axis_index("s") * NC + lax.axis_index("c")
    base = wid * rows_w

    def fire_pe(c):
        pltpu.async_copy(pe_hbm.at[pl.ds(base + c * R, R)], peb, pe_sem)

    def wait_pe(c):
        pltpu.make_async_copy(pe_hbm.at[pl.ds(base + c * R, R)], peb,
                              pe_sem).wait()

    def fire_in(c, b):
        pltpu.async_copy(x_hbm.at[b, pl.ds(base + c * R, R)], xb.at[b],
                         xin_sem.at[b])

    def wait_in(c, b):
        pltpu.make_async_copy(x_hbm.at[b, pl.ds(base + c * R, R)], xb.at[b],
                              xin_sem.at[b]).wait()

    def fire_out(c, b, q):
        pltpu.async_copy(ob.at[q], o_hbm.at[b, pl.ds(base + c * R, R)],
                         out_sem.at[q])

    def wait_out(c, b, q):
        pltpu.make_async_copy(ob.at[q], o_hbm.at[b, pl.ds(base + c * R, R)],
                              out_sem.at[q]).wait()

    fire_pe(0)
    for b in range(batch):
        fire_in(0, b)

    def outer(k, _):
        for cc in range(2):
            c = 2 * k + cc
            for b in range(batch):
                q = b % 2
                wait_in(c, b)
                if b == 0:
                    wait_pe(c)
                if b >= 2:
                    wait_out(c, b - 2, q)
                else:
                    @pl.when(c >= 1)
                    def _():
                        wait_out(c - 1, b + 2, q)

                for r in range(R):
                    @plsc.parallel_loop(0, d, step=L, unroll=8)
                    def _add(j):
                        sl = pl.ds(j, L)
                        ob[q, r, sl] = xb[b, r, sl] + peb[r, sl]

                fire_out(c, b, q)

                @pl.when(c + 1 < nch)
                def _():
                    fire_in(c + 1, b)

                if b == batch - 1:
                    @pl.when(c + 1 < nch)
                    def _():
                        fire_pe(c + 1)
        return 0

    lax.fori_loop(0, nch // 2, outer, 0)

    wait_out(nch - 1, batch - 2, (batch - 2) % 2)
    wait_out(nch - 1, batch - 1, (batch - 1) % 2)


def kernel(x, pe_table):
    batch, seq_len, d_model = x.shape
    rows_w = seq_len // NW
    nch = rows_w // R

    mesh = plsc.VectorSubcoreMesh(core_axis_name="c", subcore_axis_name="s")
    body = functools.partial(_sc_body, rows_w=rows_w, batch=batch, nch=nch,
                             d=d_model)
    return pl.kernel(
        body,
        out_type=jax.ShapeDtypeStruct(x.shape, x.dtype),
        mesh=mesh,
        scratch_types=[
            pltpu.VMEM((batch, R, d_model), jnp.float32),
            pltpu.VMEM((R, d_model), jnp.float32),
            pltpu.VMEM((2, R, d_model), jnp.float32),
            pltpu.SemaphoreType.DMA((batch,)),
            pltpu.SemaphoreType.DMA((2,)),
            pltpu.SemaphoreType.DMA,
        ],
        compiler_params=pltpu.CompilerParams(use_tc_tiling_on_sc=True),
    )(x, pe_table)


# speedup vs baseline: 1.0316x; 1.0316x over previous
import functools

import jax
import jax.numpy as jnp
from jax import lax
from jax.experimental import pallas as pl
from jax.experimental.pallas import tpu as pltpu
from jax.experimental.pallas import tpu_sc as plsc

NC, NS, L = 2, 16, 16
NW = NC * NS
R = 8


def _sc_body(x_hbm, pe_hbm, o_hbm, xb, peb, xin_sem, out_sem, pe_sem,
             *, rows_w, batch, nch, d):
    wid = lax.axis_index("s") * NC + lax.axis_index("c")
    base = wid * rows_w

    def fire_pe(c, p):
        pltpu.async_copy(pe_hbm.at[pl.ds(base + c * R, R)], peb.at[p],
                         pe_sem.at[p])

    def wait_pe(c, p):
        pltpu.make_async_copy(pe_hbm.at[pl.ds(base + c * R, R)], peb.at[p],
                              pe_sem.at[p]).wait()

    def fire_in(c, b):
        pltpu.async_copy(x_hbm.at[b, pl.ds(base + c * R, R)], xb.at[b],
                         xin_sem.at[b])

    def wait_in(c, b):
        pltpu.make_async_copy(x_hbm.at[b, pl.ds(base + c * R, R)], xb.at[b],
                              xin_sem.at[b]).wait()

    def fire_out(c, b):
        pltpu.async_copy(xb.at[b], o_hbm.at[b, pl.ds(base + c * R, R)],
                         out_sem.at[b])

    def wait_out(c, b):
        pltpu.make_async_copy(xb.at[b], o_hbm.at[b, pl.ds(base + c * R, R)],
                              out_sem.at[b]).wait()

    fire_pe(0, 0)
    fire_in(0, 0)

    def outer(k, _):
        for cc in range(2):
            c = 2 * k + cc
            for b in range(batch):
                wait_in(c, b)
                if b == 0:
                    wait_pe(c, cc)

                    @pl.when(c + 1 < nch)
                    def _():
                        fire_pe(c + 1, (cc + 1) % 2)

                if b + 1 < batch:
                    @pl.when(c >= 1)
                    def _():
                        wait_out(c - 1, b + 1)
                    fire_in(c, b + 1)
                else:
                    @pl.when(c + 1 < nch)
                    def _():
                        wait_out(c, 0)
                        fire_in(c + 1, 0)

                for r in range(R):
                    @plsc.parallel_loop(0, d, step=L, unroll=8)
                    def _add(j):
                        sl = pl.ds(j, L)
                        xb[b, r, sl] = xb[b, r, sl] + peb[cc, r, sl]

                fire_out(c, b)
        return 0

    lax.fori_loop(0, nch // 2, outer, 0)

    for b in range(batch):
        wait_out(nch - 1, b)


def kernel(x, pe_table):
    batch, seq_len, d_model = x.shape
    rows_w = seq_len // NW
    nch = rows_w // R

    mesh = plsc.VectorSubcoreMesh(core_axis_name="c", subcore_axis_name="s")
    body = functools.partial(_sc_body, rows_w=rows_w, batch=batch, nch=nch,
                             d=d_model)
    return pl.kernel(
        body,
        out_type=jax.ShapeDtypeStruct(x.shape, x.dtype),
        mesh=mesh,
        scratch_types=[
            pltpu.VMEM((batch, R, d_model), jnp.float32),
            pltpu.VMEM((2, R, d_model), jnp.float32),
            pltpu.SemaphoreType.DMA((batch,)),
            pltpu.SemaphoreType.DMA((batch,)),
            pltpu.SemaphoreType.DMA((2,)),
        ],
        compiler_params=pltpu.CompilerParams(use_tc_tiling_on_sc=True),
    )(x, pe_table)
